# baseline (speedup 1.0000x reference)
import functools

import jax
import jax.numpy as jnp
from jax import lax
from jax.experimental import pallas as pl
from jax.experimental.pallas import tpu as pltpu
from jax.experimental.pallas import tpu_sc as plsc

D_MODEL_ = 1024
BLOCK_S = 512
TC_BATCHES = 3

CHUNK_ROWS = 16
CHUNK = CHUNK_ROWS * D_MODEL_
XNBUF = 4
TNBUF = 2
PREFETCH = 3


def _tc_add_kernel(x_ref, t_ref, o_ref):
    o_ref[...] = x_ref[...] + t_ref[...][None, :, :]


def _tc_add(x, table):
    batch, seq_len, d_model = x.shape
    grid = (seq_len // BLOCK_S,)
    return pl.pallas_call(
        _tc_add_kernel,
        grid=grid,
        in_specs=[
            pl.BlockSpec((batch, BLOCK_S, d_model), lambda i: (0, i, 0)),
            pl.BlockSpec((BLOCK_S, d_model), lambda i: (i, 0)),
        ],
        out_specs=pl.BlockSpec((batch, BLOCK_S, d_model), lambda i: (0, i, 0)),
        out_shape=jax.ShapeDtypeStruct((batch, seq_len, d_model), x.dtype),
    )(x, table)


def _sc_add_kernel(x_hbm, t_hbm, o_hbm, xbuf, tbuf, semx, semt, semo,
                   *, seq_rows_per_worker, seq_len, batch):
    wid = lax.axis_index("s") * 2 + lax.axis_index("c")
    trow0 = wid * seq_rows_per_worker
    n_chunks = seq_rows_per_worker // CHUNK_ROWS
    n_units = n_chunks * batch

    def x_row(g, b):
        return b * seq_len + trow0 + g * CHUNK_ROWS

    def start_x(u, slot):
        g, b = divmod(u, batch)
        return pltpu.async_copy(
            x_hbm.at[pl.ds(x_row(g, b), CHUNK_ROWS), :], xbuf.at[slot],
            semx.at[slot])

    def start_t(g, slot):
        return pltpu.async_copy(
            t_hbm.at[pl.ds(trow0 + g * CHUNK_ROWS, CHUNK_ROWS), :],
            tbuf.at[slot], semt.at[slot])

    def start_out(u, slot):
        g, b = divmod(u, batch)
        return pltpu.async_copy(
            xbuf.at[slot], o_hbm.at[pl.ds(x_row(g, b), CHUNK_ROWS), :],
            semo.at[slot])

    pend_x = [start_x(u, u % XNBUF) for u in range(min(PREFETCH, n_units))]
    pend_x += [None] * (XNBUF - len(pend_x))
    pend_t = [start_t(g, g % TNBUF) for g in range(min(TNBUF, n_chunks))]
    pend_o = [None] * XNBUF

    for u in range(n_units):
        g, b = divmod(u, batch)
        slot = u % XNBUF
        tslot = g % TNBUF

        r = u + PREFETCH
        if r < n_units:
            rslot = r % XNBUF
            if pend_o[rslot] is not None:
                pend_o[rslot].wait()
                pend_o[rslot] = None
            pend_x[rslot] = start_x(r, rslot)

        pend_x[slot].wait()
        if b == 0:
            pend_t[tslot].wait()

        @plsc.parallel_loop(0, D_MODEL_ // 16, unroll=2)
        def add_body(i):
            s = i * 16
            for row in range(CHUNK_ROWS):
                plsc.addupdate(xbuf.at[slot, row, pl.ds(s, 16)],
                               tbuf[tslot, row, pl.ds(s, 16)])

        pend_o[slot] = start_out(u, slot)
        if b == batch - 1 and g + TNBUF < n_chunks:
            pend_t[tslot] = start_t(g + TNBUF, tslot)

    for slot in range(XNBUF):
        if pend_o[slot] is not None:
            pend_o[slot].wait()


def _sc_add(x2d, table, batch, seq_len, d_model):
    n_workers = 32
    seq_rows_per_worker = seq_len // n_workers

    mesh = plsc.VectorSubcoreMesh(core_axis_name="c", subcore_axis_name="s")
    sc_call = pl.kernel(
        functools.partial(
            _sc_add_kernel,
            seq_rows_per_worker=seq_rows_per_worker,
            seq_len=seq_len,
            batch=batch,
        ),
        mesh=mesh,
        out_type=jax.ShapeDtypeStruct((batch * seq_len, d_model), jnp.float32),
        scratch_types=[
            pltpu.VMEM((XNBUF, CHUNK_ROWS, D_MODEL_), jnp.float32),
            pltpu.VMEM((TNBUF, CHUNK_ROWS, D_MODEL_), jnp.float32),
            pltpu.SemaphoreType.DMA((XNBUF,)),
            pltpu.SemaphoreType.DMA((TNBUF,)),
            pltpu.SemaphoreType.DMA((XNBUF,)),
        ],
        compiler_params=pltpu.CompilerParams(use_tc_tiling_on_sc=True),
    )
    return sc_call(x2d, table)


def kernel(x, table):
    batch, seq_len, d_model = x.shape
    t = table[:seq_len]
    tc_out = _tc_add(x[:TC_BATCHES], t)
    sc_batches = batch - TC_BATCHES
    sc_out = _sc_add(
        x[TC_BATCHES:].reshape(sc_batches * seq_len, d_model), t,
        sc_batches, seq_len, d_model)
    return jnp.concatenate(
        [tc_out, sc_out.reshape(sc_batches, seq_len, d_model)], axis=0)


# speedup vs baseline: 3.2605x; 3.2605x over previous
import jax
import jax.numpy as jnp
from jax.experimental import pallas as pl


BLOCK_S = 512


def _add_kernel(x_ref, t_ref, o_ref):
    o_ref[...] = x_ref[...] + t_ref[...][None, :, :]


def kernel(x, table):
    batch, seq_len, d_model = x.shape
    grid = (seq_len // BLOCK_S,)
    return pl.pallas_call(
        _add_kernel,
        grid=grid,
        in_specs=[
            pl.BlockSpec((batch, BLOCK_S, d_model), lambda i: (0, i, 0)),
            pl.BlockSpec((BLOCK_S, d_model), lambda i: (i, 0)),
        ],
        out_specs=pl.BlockSpec((batch, BLOCK_S, d_model), lambda i: (0, i, 0)),
        out_shape=jax.ShapeDtypeStruct((batch, seq_len, d_model), x.dtype),
    )(x, table[:seq_len])


# speedup vs baseline: 3.2852x; 1.0076x over previous
import jax
import jax.numpy as jnp
from jax.experimental import pallas as pl


BLOCK_S = 2048


def _add_kernel(x_ref, t_ref, o_ref):
    o_ref[...] = x_ref[...] + t_ref[...][None, :, :]


def kernel(x, table):
    batch, seq_len, d_model = x.shape
    grid = (seq_len // BLOCK_S, batch)
    return pl.pallas_call(
        _add_kernel,
        grid=grid,
        in_specs=[
            pl.BlockSpec((1, BLOCK_S, d_model), lambda i, b: (b, i, 0)),
            pl.BlockSpec((BLOCK_S, d_model), lambda i, b: (i, 0)),
        ],
        out_specs=pl.BlockSpec((1, BLOCK_S, d_model), lambda i, b: (b, i, 0)),
        out_shape=jax.ShapeDtypeStruct((batch, seq_len, d_model), x.dtype),
    )(x, table[:seq_len])
